# baseline (speedup 1.0000x reference)
import functools

import jax
import jax.numpy as jnp
from jax import lax
from jax.experimental import pallas as pl
from jax.experimental.pallas import tpu as pltpu
from jax.experimental.pallas import tpu_sc as plsc

N = 10000
E = 320000
D = 128

NC = 2
NS = 16
NW = NC * NS

CHUNK = 120
CPT = 84
E_PER_TILE = CPT * CHUNK
E_PAD = NW * E_PER_TILE
N_PAD = 10240
ROWS_PER_TILE = N_PAD // NS
NBUF = 3

@functools.cache
def _mesh():
    return plsc.VectorSubcoreMesh(
        core_axis_name="c", subcore_axis_name="s", num_cores=NC, num_subcores=NS
    )


def _fill_rows(buf, nrows, width, val):
    v = jnp.full((16,), val, jnp.float32)

    def body(r, carry):
        for j in range(width // 16):
            buf[r, pl.ds(j * 16, 16)] = v
        return carry

    lax.fori_loop(0, nrows, body, 0, unroll=False)


def _zero_rows(buf, nrows, width):
    _fill_rows(buf, nrows, width, 0.0)


def _zero_acc_slice(zbuf, acc, rowbase):
    nfull, rem = divmod(ROWS_PER_TILE, CHUNK)
    for k in range(nfull):
        pltpu.sync_copy(zbuf, acc.at[pl.ds(rowbase + k * CHUNK, CHUNK)])
    if rem:
        pltpu.sync_copy(
            zbuf.at[pl.ds(0, rem)],
            acc.at[pl.ds(rowbase + nfull * CHUNK, rem)],
        )


def _deg_body(dst_hbm, deg_out, acc, d0, d1, ones_v):
    dbufs = (d0, d1)
    c = lax.axis_index("c")
    s = lax.axis_index("s")
    wid = c * NS + s
    rowbase = s * ROWS_PER_TILE

    _zero_rows(ones_v, CHUNK, D)
    _zero_acc_slice(ones_v, acc, rowbase)
    _fill_rows(ones_v, CHUNK, D, 1.0)
    plsc.subcore_barrier()

    ebase = wid * E_PER_TILE
    for b in range(2):
        pltpu.sync_copy(dst_hbm.at[pl.ds(ebase + b * CHUNK, CHUNK)], dbufs[b])

    def step(t, carry):
        for b in range(2):
            q = 2 * t + b
            pltpu.sync_copy(ones_v, acc.at[dbufs[b]], add=True)

            @pl.when(q + 2 < CPT)
            def _():
                pltpu.sync_copy(
                    dst_hbm.at[pl.ds(ebase + (q + 2) * CHUNK, CHUNK)], dbufs[b]
                )
        return carry

    lax.fori_loop(0, CPT // 2, step, 0, unroll=False)
    plsc.subcore_barrier()
    pltpu.sync_copy(
        acc.at[pl.ds(rowbase, ROWS_PER_TILE)],
        deg_out.at[c, pl.ds(rowbase, ROWS_PER_TILE)],
    )


@functools.cache
def _deg_kernel():
    return pl.kernel(
        _deg_body,
        out_type=jax.ShapeDtypeStruct((NC, N_PAD, D), jnp.float32),
        mesh=_mesh(),
        scratch_types=[
            pltpu.VMEM_SHARED((N_PAD, D), jnp.float32),
            pltpu.VMEM((CHUNK,), jnp.int32),
            pltpu.VMEM((CHUNK,), jnp.int32),
            pltpu.VMEM((CHUNK, D), jnp.float32),
        ],
    )


def _spmm_body(g_hbm, src_hbm, dst_hbm, out_hbm, acc,
               s0, s1, s2, d0, d1, d2, r0, r1, r2, m0, m1, m2):
    sbufs = (s0, s1, s2)
    dbufs = (d0, d1, d2)
    rbufs = (r0, r1, r2)
    sems = (m0, m1, m2)
    c = lax.axis_index("c")
    s = lax.axis_index("s")
    wid = c * NS + s
    rowbase = s * ROWS_PER_TILE

    _zero_rows(r0, CHUNK, D)
    _zero_acc_slice(r0, acc, rowbase)
    plsc.subcore_barrier()

    ebase = wid * E_PER_TILE

    for b in range(NBUF):
        pltpu.sync_copy(src_hbm.at[pl.ds(ebase + b * CHUNK, CHUNK)], sbufs[b])
        pltpu.sync_copy(dst_hbm.at[pl.ds(ebase + b * CHUNK, CHUNK)], dbufs[b])
        pltpu.async_copy(g_hbm.at[sbufs[b]], rbufs[b], sems[b])

    def step(t, carry):
        i = t * NBUF
        for b in range(NBUF):
            q = i + b
            pltpu.make_async_copy(g_hbm.at[sbufs[b]], rbufs[b], sems[b]).wait()
            pltpu.sync_copy(rbufs[b], acc.at[dbufs[b]], add=True)

            @pl.when(q + NBUF < CPT)
            def _():
                off = ebase + (q + NBUF) * CHUNK
                pltpu.sync_copy(src_hbm.at[pl.ds(off, CHUNK)], sbufs[b])
                pltpu.sync_copy(dst_hbm.at[pl.ds(off, CHUNK)], dbufs[b])
                pltpu.async_copy(g_hbm.at[sbufs[b]], rbufs[b], sems[b])
        return carry

    lax.fori_loop(0, CPT // NBUF, step, 0, unroll=False)
    plsc.subcore_barrier()
    pltpu.sync_copy(
        acc.at[pl.ds(rowbase, ROWS_PER_TILE)],
        out_hbm.at[c, pl.ds(rowbase, ROWS_PER_TILE)],
    )


@functools.cache
def _spmm_kernel():
    return pl.kernel(
        _spmm_body,
        out_type=jax.ShapeDtypeStruct((NC, N_PAD, D), jnp.float32),
        mesh=_mesh(),
        scratch_types=(
            [pltpu.VMEM_SHARED((N_PAD, D), jnp.float32)]
            + [pltpu.VMEM((CHUNK,), jnp.int32) for _ in range(NBUF)]
            + [pltpu.VMEM((CHUNK,), jnp.int32) for _ in range(NBUF)]
            + [pltpu.VMEM((CHUNK, D), jnp.float32) for _ in range(NBUF)]
            + [pltpu.SemaphoreType.DMA for _ in range(NBUF)]
        ),
    )


_BLK = 2560
_GRID = N_PAD // _BLK


def _mm_scale_body(degp_ref, h_ref, w_ref, g_ref, dis_ref):
    dis = lax.rsqrt(degp_ref[0, :, 0:1] + degp_ref[1, :, 0:1] + 1.0)
    dis_ref[...] = dis
    g_ref[...] = dis * jnp.dot(
        h_ref[...], w_ref[...], preferred_element_type=jnp.float32
    )


def _mm_scale(degp, h, w):
    return pl.pallas_call(
        _mm_scale_body,
        grid=(_GRID,),
        in_specs=[
            pl.BlockSpec((NC, _BLK, D), lambda i: (0, i, 0)),
            pl.BlockSpec((_BLK, D), lambda i: (i, 0)),
            pl.BlockSpec((D, D), lambda i: (0, 0)),
        ],
        out_specs=[
            pl.BlockSpec((_BLK, D), lambda i: (i, 0)),
            pl.BlockSpec((_BLK, 1), lambda i: (i, 0)),
        ],
        out_shape=[
            jax.ShapeDtypeStruct((N_PAD, D), jnp.float32),
            jax.ShapeDtypeStruct((N_PAD, 1), jnp.float32),
        ],
    )(degp, h, w)


def _combine_mm_body(p_ref, g_ref, dis_ref, b_ref, w_ref, out_ref):
    h = dis_ref[...] * (p_ref[0] + p_ref[1] + g_ref[...]) + b_ref[...]
    out_ref[...] = dis_ref[...] * jnp.dot(
        h, w_ref[...], preferred_element_type=jnp.float32
    )


def _combine_mm(p, g, dis, b, w):
    return pl.pallas_call(
        _combine_mm_body,
        grid=(_GRID,),
        in_specs=[
            pl.BlockSpec((NC, _BLK, D), lambda i: (0, i, 0)),
            pl.BlockSpec((_BLK, D), lambda i: (i, 0)),
            pl.BlockSpec((_BLK, 1), lambda i: (i, 0)),
            pl.BlockSpec((1, D), lambda i: (0, 0)),
            pl.BlockSpec((D, D), lambda i: (0, 0)),
        ],
        out_specs=pl.BlockSpec((_BLK, D), lambda i: (i, 0)),
        out_shape=jax.ShapeDtypeStruct((N_PAD, D), jnp.float32),
    )(p, g, dis, b, w)


def _final_body(p_ref, g_ref, dis_ref, b_ref, out_ref):
    out_ref[...] = (
        dis_ref[...] * (p_ref[0] + p_ref[1] + g_ref[...]) + b_ref[...]
    )


def _final(p, g, dis, b):
    return pl.pallas_call(
        _final_body,
        grid=(_GRID,),
        in_specs=[
            pl.BlockSpec((NC, _BLK, D), lambda i: (0, i, 0)),
            pl.BlockSpec((_BLK, D), lambda i: (i, 0)),
            pl.BlockSpec((_BLK, 1), lambda i: (i, 0)),
            pl.BlockSpec((1, D), lambda i: (0, 0)),
        ],
        out_specs=pl.BlockSpec((_BLK, D), lambda i: (i, 0)),
        out_shape=jax.ShapeDtypeStruct((N_PAD, D), jnp.float32),
    )(p, g, dis, b)


def kernel(x, edge_index, W_gcn, b_gcn, W_out, b_out):
    pad_iota = jnp.arange(E_PAD - E, dtype=jnp.int32)
    src = jnp.concatenate([edge_index[0], pad_iota % N])
    dst = jnp.concatenate([edge_index[1], N + pad_iota % (N_PAD - N)])
    x_pad = jnp.zeros((N_PAD, D), jnp.float32).at[:N].set(x)
    b_gcn2 = b_gcn.reshape(1, D)
    b_out2 = b_out.reshape(1, D)

    degp = _deg_kernel()(dst)
    g, dis = _mm_scale(degp, x_pad, W_gcn)
    for w_next in (W_gcn, W_gcn, W_out):
        p = _spmm_kernel()(g, src, dst)
        g = _combine_mm(p, g, dis, b_gcn2, w_next)
    p = _spmm_kernel()(g, src, dst)
    out = _final(p, g, dis, b_out2)
    return out[:N]


# speedup vs baseline: 1.2237x; 1.2237x over previous
import functools

import jax
import jax.numpy as jnp
from jax import lax
from jax.experimental import pallas as pl
from jax.experimental.pallas import tpu as pltpu
from jax.experimental.pallas import tpu_sc as plsc

N = 10000
E = 320000
D = 128

NC = 2
NS = 16
NW = NC * NS

CHUNK = 128
CPT = 80
E_PER_TILE = CPT * CHUNK
E_PAD = NW * E_PER_TILE
N_PAD = 10240
ROWS_PER_TILE = N_PAD // NS
NBUF = 2


@functools.cache
def _mesh():
    return plsc.VectorSubcoreMesh(
        core_axis_name="c", subcore_axis_name="s", num_cores=NC, num_subcores=NS
    )


def _fill_rows(buf, nrows, width, val):
    v = jnp.full((16,), val, jnp.float32)

    def body(r, carry):
        for j in range(width // 16):
            buf[r, pl.ds(j * 16, 16)] = v
        return carry

    lax.fori_loop(0, nrows, body, 0, unroll=False)


def _zero_acc_slice(zbuf, acc, rowbase):
    nfull, rem = divmod(ROWS_PER_TILE, CHUNK)
    for k in range(nfull):
        pltpu.sync_copy(zbuf, acc.at[pl.ds(rowbase + k * CHUNK, CHUNK)])
    if rem:
        pltpu.sync_copy(
            zbuf.at[pl.ds(0, rem)],
            acc.at[pl.ds(rowbase + nfull * CHUNK, rem)],
        )


def _deg_body(dst_hbm, deg_out, acc, didx, ones_v):
    c = lax.axis_index("c")
    s = lax.axis_index("s")
    wid = c * NS + s
    rowbase = s * ROWS_PER_TILE

    _fill_rows(ones_v, CHUNK, D, 0.0)
    _zero_acc_slice(ones_v, acc, rowbase)
    _fill_rows(ones_v, CHUNK, D, 1.0)
    plsc.subcore_barrier()

    pltpu.sync_copy(dst_hbm.at[pl.ds(wid * CPT, CPT)], didx)

    def step(q, carry):
        pltpu.sync_copy(ones_v, acc.at[didx.at[q]], add=True)
        return carry

    lax.fori_loop(0, CPT, step, 0, unroll=False)
    plsc.subcore_barrier()
    pltpu.sync_copy(
        acc.at[pl.ds(rowbase, ROWS_PER_TILE)],
        deg_out.at[c, pl.ds(rowbase, ROWS_PER_TILE)],
    )


@functools.cache
def _deg_kernel():
    return pl.kernel(
        _deg_body,
        out_type=jax.ShapeDtypeStruct((NC, N_PAD, D), jnp.float32),
        mesh=_mesh(),
        scratch_types=[
            pltpu.VMEM_SHARED((N_PAD, D), jnp.float32),
            pltpu.VMEM((CPT, CHUNK), jnp.int32),
            pltpu.VMEM((CHUNK, D), jnp.float32),
        ],
    )


def _spmm_body(g_hbm, src_hbm, dst_hbm, out_hbm, acc, didx,
               s0, s1, r0, r1, m0, m1):
    sbufs = (s0, s1)
    rbufs = (r0, r1)
    sems = (m0, m1)
    c = lax.axis_index("c")
    s = lax.axis_index("s")
    wid = c * NS + s
    rowbase = s * ROWS_PER_TILE

    _fill_rows(r0, CHUNK, D, 0.0)
    _zero_acc_slice(r0, acc, rowbase)
    plsc.subcore_barrier()

    cbase = wid * CPT
    pltpu.sync_copy(dst_hbm.at[pl.ds(cbase, CPT)], didx)

    for b in range(NBUF):
        pltpu.sync_copy(src_hbm.at[cbase + b], sbufs[b])
        pltpu.async_copy(g_hbm.at[sbufs[b]], rbufs[b], sems[b])

    def step(t, carry):
        i = t * NBUF
        for b in range(NBUF):
            q = i + b
            pltpu.make_async_copy(g_hbm.at[sbufs[b]], rbufs[b], sems[b]).wait()
            pltpu.sync_copy(rbufs[b], acc.at[didx.at[q]], add=True)

            @pl.when(q + NBUF < CPT)
            def _():
                pltpu.sync_copy(src_hbm.at[cbase + q + NBUF], sbufs[b])
                pltpu.async_copy(g_hbm.at[sbufs[b]], rbufs[b], sems[b])
        return carry

    lax.fori_loop(0, CPT // NBUF, step, 0, unroll=False)
    plsc.subcore_barrier()
    pltpu.sync_copy(
        acc.at[pl.ds(rowbase, ROWS_PER_TILE)],
        out_hbm.at[c, pl.ds(rowbase, ROWS_PER_TILE)],
    )


@functools.cache
def _spmm_kernel():
    return pl.kernel(
        _spmm_body,
        out_type=jax.ShapeDtypeStruct((NC, N_PAD, D), jnp.float32),
        mesh=_mesh(),
        scratch_types=[
            pltpu.VMEM_SHARED((N_PAD, D), jnp.float32),
            pltpu.VMEM((CPT, CHUNK), jnp.int32),
            pltpu.VMEM((CHUNK,), jnp.int32),
            pltpu.VMEM((CHUNK,), jnp.int32),
            pltpu.VMEM((CHUNK, D), jnp.float32),
            pltpu.VMEM((CHUNK, D), jnp.float32),
            pltpu.SemaphoreType.DMA,
            pltpu.SemaphoreType.DMA,
        ],
    )


_BLK = 2560
_GRID = N_PAD // _BLK


def _mm_scale_body(degp_ref, h_ref, w_ref, g_ref, dis_ref):
    dis = lax.rsqrt(degp_ref[0, :, 0:1] + degp_ref[1, :, 0:1] + 1.0)
    dis_ref[...] = dis
    g_ref[...] = dis * jnp.dot(
        h_ref[...], w_ref[...], preferred_element_type=jnp.float32
    )


def _mm_scale(degp, h, w):
    return pl.pallas_call(
        _mm_scale_body,
        grid=(_GRID,),
        in_specs=[
            pl.BlockSpec((NC, _BLK, D), lambda i: (0, i, 0)),
            pl.BlockSpec((_BLK, D), lambda i: (i, 0)),
            pl.BlockSpec((D, D), lambda i: (0, 0)),
        ],
        out_specs=[
            pl.BlockSpec((_BLK, D), lambda i: (i, 0)),
            pl.BlockSpec((_BLK, 1), lambda i: (i, 0)),
        ],
        out_shape=[
            jax.ShapeDtypeStruct((N_PAD, D), jnp.float32),
            jax.ShapeDtypeStruct((N_PAD, 1), jnp.float32),
        ],
    )(degp, h, w)


def _combine_mm_body(p_ref, g_ref, dis_ref, b_ref, w_ref, out_ref):
    h = dis_ref[...] * (p_ref[0] + p_ref[1] + g_ref[...]) + b_ref[...]
    out_ref[...] = dis_ref[...] * jnp.dot(
        h, w_ref[...], preferred_element_type=jnp.float32
    )


def _combine_mm(p, g, dis, b, w):
    return pl.pallas_call(
        _combine_mm_body,
        grid=(_GRID,),
        in_specs=[
            pl.BlockSpec((NC, _BLK, D), lambda i: (0, i, 0)),
            pl.BlockSpec((_BLK, D), lambda i: (i, 0)),
            pl.BlockSpec((_BLK, 1), lambda i: (i, 0)),
            pl.BlockSpec((1, D), lambda i: (0, 0)),
            pl.BlockSpec((D, D), lambda i: (0, 0)),
        ],
        out_specs=pl.BlockSpec((_BLK, D), lambda i: (i, 0)),
        out_shape=jax.ShapeDtypeStruct((N_PAD, D), jnp.float32),
    )(p, g, dis, b, w)


def _final_body(p_ref, g_ref, dis_ref, b_ref, out_ref):
    out_ref[...] = (
        dis_ref[...] * (p_ref[0] + p_ref[1] + g_ref[...]) + b_ref[...]
    )


def _final(p, g, dis, b):
    return pl.pallas_call(
        _final_body,
        grid=(_GRID,),
        in_specs=[
            pl.BlockSpec((NC, _BLK, D), lambda i: (0, i, 0)),
            pl.BlockSpec((_BLK, D), lambda i: (i, 0)),
            pl.BlockSpec((_BLK, 1), lambda i: (i, 0)),
            pl.BlockSpec((1, D), lambda i: (0, 0)),
        ],
        out_specs=pl.BlockSpec((_BLK, D), lambda i: (i, 0)),
        out_shape=jax.ShapeDtypeStruct((N_PAD, D), jnp.float32),
    )(p, g, dis, b)


def kernel(x, edge_index, W_gcn, b_gcn, W_out, b_out):
    pad_iota = jnp.arange(E_PAD - E, dtype=jnp.int32)
    src = jnp.concatenate([edge_index[0], pad_iota % N])
    dst = jnp.concatenate([edge_index[1], N + pad_iota % (N_PAD - N)])
    src2d = src.reshape(E_PAD // CHUNK, CHUNK)
    dst2d = dst.reshape(E_PAD // CHUNK, CHUNK)
    x_pad = jnp.zeros((N_PAD, D), jnp.float32).at[:N].set(x)
    b_gcn2 = b_gcn.reshape(1, D)
    b_out2 = b_out.reshape(1, D)

    degp = _deg_kernel()(dst2d)
    g, dis = _mm_scale(degp, x_pad, W_gcn)
    for w_next in (W_gcn, W_gcn, W_out):
        p = _spmm_kernel()(g, src2d, dst2d)
        g = _combine_mm(p, g, dis, b_gcn2, w_next)
    p = _spmm_kernel()(g, src2d, dst2d)
    out = _final(p, g, dis, b_out2)
    return out[:N]


# speedup vs baseline: 1.3491x; 1.1026x over previous
import functools

import jax
import jax.numpy as jnp
from jax import lax
from jax.experimental import pallas as pl
from jax.experimental.pallas import tpu as pltpu
from jax.experimental.pallas import tpu_sc as plsc

N = 10000
E = 320000
D = 128

NC = 2
NS = 16
NW = NC * NS

CHUNK = 128
CPT = 80
E_PER_TILE = CPT * CHUNK
E_PAD = NW * E_PER_TILE
N_PAD = 10240
ROWS_PER_TILE = N_PAD // NS
NBUF = 2


@functools.cache
def _mesh():
    return plsc.VectorSubcoreMesh(
        core_axis_name="c", subcore_axis_name="s", num_cores=NC, num_subcores=NS
    )


def _fill_rows(buf, nrows, width, val):
    v = jnp.full((16,), val, jnp.float32)

    def body(r, carry):
        for j in range(width // 16):
            buf[r, pl.ds(j * 16, 16)] = v
        return carry

    lax.fori_loop(0, nrows, body, 0, unroll=False)


def _zero_acc_slice(zbuf, acc, rowbase):
    nfull, rem = divmod(ROWS_PER_TILE, CHUNK)
    for k in range(nfull):
        pltpu.sync_copy(zbuf, acc.at[pl.ds(rowbase + k * CHUNK, CHUNK)])
    if rem:
        pltpu.sync_copy(
            zbuf.at[pl.ds(0, rem)],
            acc.at[pl.ds(rowbase + nfull * CHUNK, rem)],
        )


def _deg_body(dst_hbm, deg_out, acc, didx, ones_v):
    c = lax.axis_index("c")
    s = lax.axis_index("s")
    wid = c * NS + s
    rowbase = s * ROWS_PER_TILE

    _fill_rows(ones_v, CHUNK, D, 0.0)
    _zero_acc_slice(ones_v, acc, rowbase)
    _fill_rows(ones_v, CHUNK, D, 1.0)
    plsc.subcore_barrier()

    pltpu.sync_copy(dst_hbm.at[pl.ds(wid * CPT, CPT)], didx)

    def step(q, carry):
        pltpu.sync_copy(ones_v, acc.at[didx.at[q]], add=True)
        return carry

    lax.fori_loop(0, CPT, step, 0, unroll=False)
    plsc.subcore_barrier()
    pltpu.sync_copy(
        acc.at[pl.ds(rowbase, ROWS_PER_TILE)],
        deg_out.at[c, pl.ds(rowbase, ROWS_PER_TILE)],
    )


@functools.cache
def _deg_kernel():
    return pl.kernel(
        _deg_body,
        out_type=jax.ShapeDtypeStruct((NC, N_PAD, D), jnp.float32),
        mesh=_mesh(),
        scratch_types=[
            pltpu.VMEM_SHARED((N_PAD, D), jnp.float32),
            pltpu.VMEM((CPT, CHUNK), jnp.int32),
            pltpu.VMEM((CHUNK, D), jnp.float32),
        ],
    )


def _spmm_body(g_hbm, src_hbm, dst_hbm, out_hbm, acc, didx,
               s0, s1, r0, r1, m0, m1, isem):
    sbufs = (s0, s1)
    rbufs = (r0, r1)
    sems = (m0, m1)
    c = lax.axis_index("c")
    s = lax.axis_index("s")
    wid = c * NS + s
    rowbase = s * ROWS_PER_TILE

    _fill_rows(r0, CHUNK, D, 0.0)
    _zero_acc_slice(r0, acc, rowbase)
    plsc.subcore_barrier()

    cbase = wid * CPT
    pltpu.sync_copy(dst_hbm.at[pl.ds(cbase, CPT)], didx)

    for b in range(NBUF):
        pltpu.sync_copy(src_hbm.at[cbase + b], sbufs[b])
        pltpu.async_copy(g_hbm.at[sbufs[b]], rbufs[b], sems[b])

    def step(t, carry):
        i = t * NBUF
        for b in range(NBUF):
            q = i + b
            pltpu.make_async_copy(g_hbm.at[sbufs[b]], rbufs[b], sems[b]).wait()
            nxt = pltpu.async_copy(src_hbm.at[cbase + q + NBUF], sbufs[b], isem)
            pltpu.sync_copy(rbufs[b], acc.at[didx.at[q]], add=True)
            nxt.wait()
            pltpu.async_copy(g_hbm.at[sbufs[b]], rbufs[b], sems[b])
        return carry

    lax.fori_loop(0, CPT // NBUF - 1, step, 0, unroll=False)
    for b in range(NBUF):
        q = CPT - NBUF + b
        pltpu.make_async_copy(g_hbm.at[sbufs[b]], rbufs[b], sems[b]).wait()
        pltpu.sync_copy(rbufs[b], acc.at[didx.at[q]], add=True)
    plsc.subcore_barrier()
    pltpu.sync_copy(
        acc.at[pl.ds(rowbase, ROWS_PER_TILE)],
        out_hbm.at[c, pl.ds(rowbase, ROWS_PER_TILE)],
    )


@functools.cache
def _spmm_kernel():
    return pl.kernel(
        _spmm_body,
        out_type=jax.ShapeDtypeStruct((NC, N_PAD, D), jnp.float32),
        mesh=_mesh(),
        scratch_types=[
            pltpu.VMEM_SHARED((N_PAD, D), jnp.float32),
            pltpu.VMEM((CPT, CHUNK), jnp.int32),
            pltpu.VMEM((CHUNK,), jnp.int32),
            pltpu.VMEM((CHUNK,), jnp.int32),
            pltpu.VMEM((CHUNK, D), jnp.float32),
            pltpu.VMEM((CHUNK, D), jnp.float32),
            pltpu.SemaphoreType.DMA,
            pltpu.SemaphoreType.DMA,
            pltpu.SemaphoreType.DMA,
        ],
    )


_BLK = 2560
_GRID = N_PAD // _BLK


def _mm_scale_body(degp_ref, h_ref, w_ref, g_ref, dis_ref):
    dis = lax.rsqrt(degp_ref[0, :, 0:1] + degp_ref[1, :, 0:1] + 1.0)
    dis_ref[...] = dis
    g_ref[...] = dis * jnp.dot(
        h_ref[...], w_ref[...], preferred_element_type=jnp.float32
    )


def _mm_scale(degp, h, w):
    return pl.pallas_call(
        _mm_scale_body,
        grid=(_GRID,),
        in_specs=[
            pl.BlockSpec((NC, _BLK, D), lambda i: (0, i, 0)),
            pl.BlockSpec((_BLK, D), lambda i: (i, 0)),
            pl.BlockSpec((D, D), lambda i: (0, 0)),
        ],
        out_specs=[
            pl.BlockSpec((_BLK, D), lambda i: (i, 0)),
            pl.BlockSpec((_BLK, 1), lambda i: (i, 0)),
        ],
        out_shape=[
            jax.ShapeDtypeStruct((N_PAD, D), jnp.float32),
            jax.ShapeDtypeStruct((N_PAD, 1), jnp.float32),
        ],
    )(degp, h, w)


def _combine_mm_body(p_ref, g_ref, dis_ref, b_ref, w_ref, out_ref):
    h = dis_ref[...] * (p_ref[0] + p_ref[1] + g_ref[...]) + b_ref[...]
    out_ref[...] = dis_ref[...] * jnp.dot(
        h, w_ref[...], preferred_element_type=jnp.float32
    )


def _combine_mm(p, g, dis, b, w):
    return pl.pallas_call(
        _combine_mm_body,
        grid=(_GRID,),
        in_specs=[
            pl.BlockSpec((NC, _BLK, D), lambda i: (0, i, 0)),
            pl.BlockSpec((_BLK, D), lambda i: (i, 0)),
            pl.BlockSpec((_BLK, 1), lambda i: (i, 0)),
            pl.BlockSpec((1, D), lambda i: (0, 0)),
            pl.BlockSpec((D, D), lambda i: (0, 0)),
        ],
        out_specs=pl.BlockSpec((_BLK, D), lambda i: (i, 0)),
        out_shape=jax.ShapeDtypeStruct((N_PAD, D), jnp.float32),
    )(p, g, dis, b, w)


def _final_body(p_ref, g_ref, dis_ref, b_ref, out_ref):
    out_ref[...] = (
        dis_ref[...] * (p_ref[0] + p_ref[1] + g_ref[...]) + b_ref[...]
    )


def _final(p, g, dis, b):
    return pl.pallas_call(
        _final_body,
        grid=(_GRID,),
        in_specs=[
            pl.BlockSpec((NC, _BLK, D), lambda i: (0, i, 0)),
            pl.BlockSpec((_BLK, D), lambda i: (i, 0)),
            pl.BlockSpec((_BLK, 1), lambda i: (i, 0)),
            pl.BlockSpec((1, D), lambda i: (0, 0)),
        ],
        out_specs=pl.BlockSpec((_BLK, D), lambda i: (i, 0)),
        out_shape=jax.ShapeDtypeStruct((N_PAD, D), jnp.float32),
    )(p, g, dis, b)


def kernel(x, edge_index, W_gcn, b_gcn, W_out, b_out):
    pad_iota = jnp.arange(E_PAD - E, dtype=jnp.int32)
    src = jnp.concatenate([edge_index[0], pad_iota % N])
    dst = jnp.concatenate([edge_index[1], N + pad_iota % (N_PAD - N)])
    src2d = src.reshape(E_PAD // CHUNK, CHUNK)
    dst2d = dst.reshape(E_PAD // CHUNK, CHUNK)
    x_pad = jnp.zeros((N_PAD, D), jnp.float32).at[:N].set(x)
    b_gcn2 = b_gcn.reshape(1, D)
    b_out2 = b_out.reshape(1, D)

    degp = _deg_kernel()(dst2d)
    g, dis = _mm_scale(degp, x_pad, W_gcn)
    for w_next in (W_gcn, W_gcn, W_out):
        p = _spmm_kernel()(g, src2d, dst2d)
        g = _combine_mm(p, g, dis, b_gcn2, w_next)
    p = _spmm_kernel()(g, src2d, dst2d)
    out = _final(p, g, dis, b_out2)
    return out[:N]


# speedup vs baseline: 1.3511x; 1.0014x over previous
import functools

import jax
import jax.numpy as jnp
from jax import lax
from jax.experimental import pallas as pl
from jax.experimental.pallas import tpu as pltpu
from jax.experimental.pallas import tpu_sc as plsc

N = 10000
E = 320000
D = 128

NC = 2
NS = 16
NW = NC * NS

CHUNK = 128
CPT = 80
E_PER_TILE = CPT * CHUNK
E_PAD = NW * E_PER_TILE
N_PAD = 10240
ROWS_PER_TILE = N_PAD // NS
NBUF = 2


@functools.cache
def _mesh():
    return plsc.VectorSubcoreMesh(
        core_axis_name="c", subcore_axis_name="s", num_cores=NC, num_subcores=NS
    )


def _fill_rows(buf, nrows, width, val):
    v = jnp.full((16,), val, jnp.float32)

    def body(r, carry):
        for j in range(width // 16):
            buf[r, pl.ds(j * 16, 16)] = v
        return carry

    lax.fori_loop(0, nrows, body, 0, unroll=False)


def _zero_acc_slice(zbuf, acc, rowbase):
    nfull, rem = divmod(ROWS_PER_TILE, CHUNK)
    for k in range(nfull):
        pltpu.sync_copy(zbuf, acc.at[pl.ds(rowbase + k * CHUNK, CHUNK)])
    if rem:
        pltpu.sync_copy(
            zbuf.at[pl.ds(0, rem)],
            acc.at[pl.ds(rowbase + nfull * CHUNK, rem)],
        )


def _deg_body(dst_hbm, deg_out, acc, didx, ones_v):
    c = lax.axis_index("c")
    s = lax.axis_index("s")
    wid = c * NS + s
    rowbase = s * ROWS_PER_TILE

    _fill_rows(ones_v, CHUNK, D, 0.0)
    _zero_acc_slice(ones_v, acc, rowbase)
    _fill_rows(ones_v, CHUNK, D, 1.0)
    plsc.subcore_barrier()

    pltpu.sync_copy(dst_hbm.at[pl.ds(wid * CPT, CPT)], didx)

    def step(q, carry):
        pltpu.sync_copy(ones_v, acc.at[didx.at[q]], add=True)
        return carry

    lax.fori_loop(0, CPT, step, 0, unroll=False)
    plsc.subcore_barrier()
    pltpu.sync_copy(
        acc.at[pl.ds(rowbase, ROWS_PER_TILE)],
        deg_out.at[c, pl.ds(rowbase, ROWS_PER_TILE)],
    )


@functools.cache
def _deg_kernel():
    return pl.kernel(
        _deg_body,
        out_type=jax.ShapeDtypeStruct((NC, N_PAD, D), jnp.float32),
        mesh=_mesh(),
        scratch_types=[
            pltpu.VMEM_SHARED((N_PAD, D), jnp.float32),
            pltpu.VMEM((CPT, CHUNK), jnp.int32),
            pltpu.VMEM((CHUNK, D), jnp.float32),
        ],
    )


def _spmm_body(g_hbm, src_hbm, dst_hbm, out_hbm, acc, didx,
               s0, s1, r0, r1, m0, m1, isem):
    sbufs = (s0, s1)
    rbufs = (r0, r1)
    sems = (m0, m1)
    c = lax.axis_index("c")
    s = lax.axis_index("s")
    wid = c * NS + s
    rowbase = s * ROWS_PER_TILE

    _fill_rows(r0, CHUNK, D, 0.0)
    _zero_acc_slice(r0, acc, rowbase)
    plsc.subcore_barrier()

    cbase = wid * CPT
    pltpu.sync_copy(dst_hbm.at[pl.ds(cbase, CPT)], didx)

    for b in range(NBUF):
        pltpu.sync_copy(src_hbm.at[cbase + b], sbufs[b])
        pltpu.async_copy(g_hbm.at[sbufs[b]], rbufs[b], sems[b])

    def step(t, carry):
        i = t * NBUF
        for b in range(NBUF):
            q = i + b
            pltpu.make_async_copy(g_hbm.at[sbufs[b]], rbufs[b], sems[b]).wait()
            nxt = pltpu.async_copy(src_hbm.at[cbase + q + NBUF], sbufs[b], isem)
            pltpu.sync_copy(rbufs[b], acc.at[didx.at[q]], add=True)
            nxt.wait()
            pltpu.async_copy(g_hbm.at[sbufs[b]], rbufs[b], sems[b])
        return carry

    lax.fori_loop(0, CPT // NBUF - 1, step, 0, unroll=False)
    for b in range(NBUF):
        q = CPT - NBUF + b
        pltpu.make_async_copy(g_hbm.at[sbufs[b]], rbufs[b], sems[b]).wait()
        pltpu.sync_copy(rbufs[b], acc.at[didx.at[q]], add=True)
    plsc.subcore_barrier()
    pltpu.sync_copy(
        acc.at[pl.ds(rowbase, ROWS_PER_TILE)],
        out_hbm.at[c, pl.ds(rowbase, ROWS_PER_TILE)],
    )


@functools.cache
def _spmm_kernel():
    return pl.kernel(
        _spmm_body,
        out_type=jax.ShapeDtypeStruct((NC, N_PAD, D), jnp.float32),
        mesh=_mesh(),
        scratch_types=[
            pltpu.VMEM_SHARED((N_PAD, D), jnp.float32),
            pltpu.VMEM((CPT, CHUNK), jnp.int32),
            pltpu.VMEM((CHUNK,), jnp.int32),
            pltpu.VMEM((CHUNK,), jnp.int32),
            pltpu.VMEM((CHUNK, D), jnp.float32),
            pltpu.VMEM((CHUNK, D), jnp.float32),
            pltpu.SemaphoreType.DMA,
            pltpu.SemaphoreType.DMA,
            pltpu.SemaphoreType.DMA,
        ],
    )


_BLK = 5120
_GRID = N_PAD // _BLK


def _mm_scale_body(degp_ref, h_ref, w_ref, g_ref, dis_ref):
    dis = lax.rsqrt(degp_ref[0, :, 0:1] + degp_ref[1, :, 0:1] + 1.0)
    dis_ref[...] = dis
    g_ref[...] = dis * jnp.dot(
        h_ref[...], w_ref[...], preferred_element_type=jnp.float32
    )


def _mm_scale(degp, h, w):
    return pl.pallas_call(
        _mm_scale_body,
        grid=(_GRID,),
        in_specs=[
            pl.BlockSpec((NC, _BLK, D), lambda i: (0, i, 0)),
            pl.BlockSpec((_BLK, D), lambda i: (i, 0)),
            pl.BlockSpec((D, D), lambda i: (0, 0)),
        ],
        out_specs=[
            pl.BlockSpec((_BLK, D), lambda i: (i, 0)),
            pl.BlockSpec((_BLK, 1), lambda i: (i, 0)),
        ],
        out_shape=[
            jax.ShapeDtypeStruct((N_PAD, D), jnp.float32),
            jax.ShapeDtypeStruct((N_PAD, 1), jnp.float32),
        ],
    )(degp, h, w)


def _combine_mm_body(p_ref, g_ref, dis_ref, b_ref, w_ref, out_ref):
    h = dis_ref[...] * (p_ref[0] + p_ref[1] + g_ref[...]) + b_ref[...]
    out_ref[...] = dis_ref[...] * jnp.dot(
        h, w_ref[...], preferred_element_type=jnp.float32
    )


def _combine_mm(p, g, dis, b, w):
    return pl.pallas_call(
        _combine_mm_body,
        grid=(_GRID,),
        in_specs=[
            pl.BlockSpec((NC, _BLK, D), lambda i: (0, i, 0)),
            pl.BlockSpec((_BLK, D), lambda i: (i, 0)),
            pl.BlockSpec((_BLK, 1), lambda i: (i, 0)),
            pl.BlockSpec((1, D), lambda i: (0, 0)),
            pl.BlockSpec((D, D), lambda i: (0, 0)),
        ],
        out_specs=pl.BlockSpec((_BLK, D), lambda i: (i, 0)),
        out_shape=jax.ShapeDtypeStruct((N_PAD, D), jnp.float32),
    )(p, g, dis, b, w)


def _final_body(p_ref, g_ref, dis_ref, b_ref, out_ref):
    out_ref[...] = (
        dis_ref[...] * (p_ref[0] + p_ref[1] + g_ref[...]) + b_ref[...]
    )


def _final(p, g, dis, b):
    return pl.pallas_call(
        _final_body,
        grid=(_GRID,),
        in_specs=[
            pl.BlockSpec((NC, _BLK, D), lambda i: (0, i, 0)),
            pl.BlockSpec((_BLK, D), lambda i: (i, 0)),
            pl.BlockSpec((_BLK, 1), lambda i: (i, 0)),
            pl.BlockSpec((1, D), lambda i: (0, 0)),
        ],
        out_specs=pl.BlockSpec((_BLK, D), lambda i: (i, 0)),
        out_shape=jax.ShapeDtypeStruct((N_PAD, D), jnp.float32),
    )(p, g, dis, b)


def kernel(x, edge_index, W_gcn, b_gcn, W_out, b_out):
    pad_iota = jnp.arange(E_PAD - E, dtype=jnp.int32)
    src = jnp.concatenate([edge_index[0], pad_iota % N])
    dst = jnp.concatenate([edge_index[1], N + pad_iota % (N_PAD - N)])
    src2d = src.reshape(E_PAD // CHUNK, CHUNK)
    dst2d = dst.reshape(E_PAD // CHUNK, CHUNK)
    x_pad = jnp.zeros((N_PAD, D), jnp.float32).at[:N].set(x)
    b_gcn2 = b_gcn.reshape(1, D)
    b_out2 = b_out.reshape(1, D)

    degp = _deg_kernel()(dst2d)
    g, dis = _mm_scale(degp, x_pad, W_gcn)
    for w_next in (W_gcn, W_gcn, W_out):
        p = _spmm_kernel()(g, src2d, dst2d)
        g = _combine_mm(p, g, dis, b_gcn2, w_next)
    p = _spmm_kernel()(g, src2d, dst2d)
    out = _final(p, g, dis, b_out2)
    return out[:N]


# speedup vs baseline: 1.3570x; 1.0044x over previous
import functools

import jax
import jax.numpy as jnp
from jax import lax
from jax.experimental import pallas as pl
from jax.experimental.pallas import tpu as pltpu
from jax.experimental.pallas import tpu_sc as plsc

N = 10000
E = 320000
D = 128

NC = 2
NS = 16
NW = NC * NS

CHUNK = 128
CPT = 80
E_PER_TILE = CPT * CHUNK
E_PAD = NW * E_PER_TILE
N_PAD = 10240
ROWS_PER_TILE = N_PAD // NS
NBUF = 2


@functools.cache
def _mesh():
    return plsc.VectorSubcoreMesh(
        core_axis_name="c", subcore_axis_name="s", num_cores=NC, num_subcores=NS
    )


def _fill_rows(buf, nrows, width, val):
    v = jnp.full((16,), val, jnp.float32)

    def body(r, carry):
        for j in range(width // 16):
            buf[r, pl.ds(j * 16, 16)] = v
        return carry

    lax.fori_loop(0, nrows, body, 0, unroll=False)


def _zero_acc_slice(zbuf, acc, rowbase):
    nfull, rem = divmod(ROWS_PER_TILE, CHUNK)
    for k in range(nfull):
        pltpu.sync_copy(zbuf, acc.at[pl.ds(rowbase + k * CHUNK, CHUNK)])
    if rem:
        pltpu.sync_copy(
            zbuf.at[pl.ds(0, rem)],
            acc.at[pl.ds(rowbase + nfull * CHUNK, rem)],
        )


def _deg_body(dst_hbm, deg_out, acc, didx, ones_v):
    c = lax.axis_index("c")
    s = lax.axis_index("s")
    wid = c * NS + s
    rowbase = s * ROWS_PER_TILE

    _fill_rows(ones_v, CHUNK, D, 0.0)
    _zero_acc_slice(ones_v, acc, rowbase)
    _fill_rows(ones_v, CHUNK, D, 1.0)
    plsc.subcore_barrier()

    pltpu.sync_copy(dst_hbm.at[pl.ds(wid * CPT, CPT)], didx)

    def step(q, carry):
        pltpu.sync_copy(ones_v, acc.at[didx.at[q]], add=True)
        return carry

    lax.fori_loop(0, CPT, step, 0, unroll=False)
    plsc.subcore_barrier()
    pltpu.sync_copy(
        acc.at[pl.ds(rowbase, ROWS_PER_TILE)],
        deg_out.at[c, pl.ds(rowbase, ROWS_PER_TILE)],
    )


@functools.cache
def _deg_kernel():
    return pl.kernel(
        _deg_body,
        out_type=jax.ShapeDtypeStruct((NC, N_PAD, D), jnp.float32),
        mesh=_mesh(),
        scratch_types=[
            pltpu.VMEM_SHARED((N_PAD, D), jnp.float32),
            pltpu.VMEM((CPT, CHUNK), jnp.int32),
            pltpu.VMEM((CHUNK, D), jnp.float32),
        ],
    )


def _spmm_body(g_hbm, src_hbm, dst_hbm, out_hbm, acc, didx,
               s0, s1, r0, r1, m0, m1, isem):
    sbufs = (s0, s1)
    rbufs = (r0, r1)
    sems = (m0, m1)
    c = lax.axis_index("c")
    s = lax.axis_index("s")
    wid = c * NS + s
    rowbase = s * ROWS_PER_TILE

    cbase = wid * CPT
    _fill_rows(r0, CHUNK, D, 0.0)
    pltpu.sync_copy(src_hbm.at[cbase + 1], sbufs[1])
    pltpu.async_copy(g_hbm.at[sbufs[1]], rbufs[1], sems[1])
    _zero_acc_slice(r0, acc, rowbase)
    pltpu.sync_copy(dst_hbm.at[pl.ds(cbase, CPT)], didx)
    pltpu.sync_copy(src_hbm.at[cbase + 0], sbufs[0])
    pltpu.async_copy(g_hbm.at[sbufs[0]], rbufs[0], sems[0])
    plsc.subcore_barrier()

    def step(t, carry):
        i = t * NBUF
        for b in range(NBUF):
            q = i + b
            pltpu.make_async_copy(g_hbm.at[sbufs[b]], rbufs[b], sems[b]).wait()
            nxt = pltpu.async_copy(src_hbm.at[cbase + q + NBUF], sbufs[b], isem)
            pltpu.sync_copy(rbufs[b], acc.at[didx.at[q]], add=True)
            nxt.wait()
            pltpu.async_copy(g_hbm.at[sbufs[b]], rbufs[b], sems[b])
        return carry

    lax.fori_loop(0, CPT // NBUF - 1, step, 0, unroll=False)
    for b in range(NBUF):
        q = CPT - NBUF + b
        pltpu.make_async_copy(g_hbm.at[sbufs[b]], rbufs[b], sems[b]).wait()
        pltpu.sync_copy(rbufs[b], acc.at[didx.at[q]], add=True)
    plsc.subcore_barrier()
    pltpu.sync_copy(
        acc.at[pl.ds(rowbase, ROWS_PER_TILE)],
        out_hbm.at[c, pl.ds(rowbase, ROWS_PER_TILE)],
    )


@functools.cache
def _spmm_kernel():
    return pl.kernel(
        _spmm_body,
        out_type=jax.ShapeDtypeStruct((NC, N_PAD, D), jnp.float32),
        mesh=_mesh(),
        scratch_types=[
            pltpu.VMEM_SHARED((N_PAD, D), jnp.float32),
            pltpu.VMEM((CPT, CHUNK), jnp.int32),
            pltpu.VMEM((CHUNK,), jnp.int32),
            pltpu.VMEM((CHUNK,), jnp.int32),
            pltpu.VMEM((CHUNK, D), jnp.float32),
            pltpu.VMEM((CHUNK, D), jnp.float32),
            pltpu.SemaphoreType.DMA,
            pltpu.SemaphoreType.DMA,
            pltpu.SemaphoreType.DMA,
        ],
    )


_BLK = 5120
_GRID = N_PAD // _BLK


def _mm_scale_body(degp_ref, h_ref, w_ref, g_ref, dis_ref):
    dis = lax.rsqrt(degp_ref[0, :, 0:1] + degp_ref[1, :, 0:1] + 1.0)
    dis_ref[...] = dis
    g_ref[...] = dis * jnp.dot(
        h_ref[...], w_ref[...], preferred_element_type=jnp.float32
    )


def _mm_scale(degp, h, w):
    return pl.pallas_call(
        _mm_scale_body,
        grid=(_GRID,),
        in_specs=[
            pl.BlockSpec((NC, _BLK, D), lambda i: (0, i, 0)),
            pl.BlockSpec((_BLK, D), lambda i: (i, 0)),
            pl.BlockSpec((D, D), lambda i: (0, 0)),
        ],
        out_specs=[
            pl.BlockSpec((_BLK, D), lambda i: (i, 0)),
            pl.BlockSpec((_BLK, 1), lambda i: (i, 0)),
        ],
        out_shape=[
            jax.ShapeDtypeStruct((N_PAD, D), jnp.float32),
            jax.ShapeDtypeStruct((N_PAD, 1), jnp.float32),
        ],
    )(degp, h, w)


def _combine_mm_body(p_ref, g_ref, dis_ref, b_ref, w_ref, out_ref):
    h = dis_ref[...] * (p_ref[0] + p_ref[1] + g_ref[...]) + b_ref[...]
    out_ref[...] = dis_ref[...] * jnp.dot(
        h, w_ref[...], preferred_element_type=jnp.float32
    )


def _combine_mm(p, g, dis, b, w):
    return pl.pallas_call(
        _combine_mm_body,
        grid=(_GRID,),
        in_specs=[
            pl.BlockSpec((NC, _BLK, D), lambda i: (0, i, 0)),
            pl.BlockSpec((_BLK, D), lambda i: (i, 0)),
            pl.BlockSpec((_BLK, 1), lambda i: (i, 0)),
            pl.BlockSpec((1, D), lambda i: (0, 0)),
            pl.BlockSpec((D, D), lambda i: (0, 0)),
        ],
        out_specs=pl.BlockSpec((_BLK, D), lambda i: (i, 0)),
        out_shape=jax.ShapeDtypeStruct((N_PAD, D), jnp.float32),
    )(p, g, dis, b, w)


def _final_body(p_ref, g_ref, dis_ref, b_ref, out_ref):
    out_ref[...] = (
        dis_ref[...] * (p_ref[0] + p_ref[1] + g_ref[...]) + b_ref[...]
    )


def _final(p, g, dis, b):
    return pl.pallas_call(
        _final_body,
        grid=(_GRID,),
        in_specs=[
            pl.BlockSpec((NC, _BLK, D), lambda i: (0, i, 0)),
            pl.BlockSpec((_BLK, D), lambda i: (i, 0)),
            pl.BlockSpec((_BLK, 1), lambda i: (i, 0)),
            pl.BlockSpec((1, D), lambda i: (0, 0)),
        ],
        out_specs=pl.BlockSpec((_BLK, D), lambda i: (i, 0)),
        out_shape=jax.ShapeDtypeStruct((N, D), jnp.float32),
    )(p, g, dis, b)


def kernel(x, edge_index, W_gcn, b_gcn, W_out, b_out):
    pad_iota = jnp.arange(E_PAD - E, dtype=jnp.int32)
    src = jnp.concatenate([edge_index[0], pad_iota % N])
    dst = jnp.concatenate([edge_index[1], N + pad_iota % (N_PAD - N)])
    src2d = src.reshape(E_PAD // CHUNK, CHUNK)
    dst2d = dst.reshape(E_PAD // CHUNK, CHUNK)
    x_pad = jnp.zeros((N_PAD, D), jnp.float32).at[:N].set(x)
    b_gcn2 = b_gcn.reshape(1, D)
    b_out2 = b_out.reshape(1, D)

    degp = _deg_kernel()(dst2d)
    g, dis = _mm_scale(degp, x_pad, W_gcn)
    for w_next in (W_gcn, W_gcn, W_out):
        p = _spmm_kernel()(g, src2d, dst2d)
        g = _combine_mm(p, g, dis, b_gcn2, w_next)
    p = _spmm_kernel()(g, src2d, dst2d)
    return _final(p, g, dis, b_out2)
